# baseline (speedup 1.0000x reference)
import functools

import jax
import jax.numpy as jnp
from jax.experimental import pallas as pl
from jax.experimental.pallas import tpu as pltpu

_BN_EPS = 1e-5


def _make_kernel(n, block_m, nbuf, out_chunks):
    nblk = n // block_m
    blk_per_chunk = nblk // out_chunks
    chunk_rows = n // out_chunks

    def _kernel(hf_ref, w_ref, gamma_ref, beta_ref, a_hbm, out_hbm,
                ht_scratch, abuf, out_vmem, sems, out_sems):
        def copy_in(i):
            return pltpu.make_async_copy(
                a_hbm.at[pl.ds(i * block_m, block_m), :],
                abuf.at[i % nbuf],
                sems.at[i % nbuf])

        def copy_out(c):
            rows = pl.ds(c * chunk_rows, chunk_rows)
            return pltpu.make_async_copy(
                out_vmem.at[rows, :], out_hbm.at[rows, :], out_sems.at[c])

        for i in range(min(nbuf, nblk)):
            copy_in(i).start()

        ht = jnp.dot(hf_ref[...], w_ref[...],
                     preferred_element_type=jnp.float32)
        mean = jnp.mean(ht, axis=0, keepdims=True)
        var = jnp.mean(jnp.square(ht - mean), axis=0, keepdims=True)
        inv = jax.lax.rsqrt(var + _BN_EPS)
        htn = (ht - mean) * inv * gamma_ref[...] + beta_ref[...]
        ht_scratch[...] = jnp.maximum(htn, 0.0)

        for i in range(nblk):
            copy_in(i).wait()
            out_vmem[pl.ds(i * block_m, block_m), :] = jnp.dot(
                abuf[i % nbuf], ht_scratch[...],
                preferred_element_type=jnp.float32)
            if i + nbuf < nblk:
                copy_in(i + nbuf).start()
            if (i + 1) % blk_per_chunk == 0:
                copy_out((i + 1) // blk_per_chunk - 1).start()

        for c in range(out_chunks):
            copy_out(c).wait()

    return _kernel


@functools.partial(jax.jit,
                   static_argnames=("block_m", "nbuf", "out_chunks",
                                    "interpret"))
def _gcn(H, A_normalized, W, bn_gamma, bn_beta, block_m=80, nbuf=12,
         out_chunks=5, interpret=False):
    batch, n, in_dim = H.shape
    out_dim = W.shape[1]
    hf = H.reshape(batch * n, in_dim)
    gamma = bn_gamma.reshape(1, out_dim)
    beta = bn_beta.reshape(1, out_dim)

    out = pl.pallas_call(
        _make_kernel(n, block_m, nbuf, out_chunks),
        in_specs=[
            pl.BlockSpec((batch * n, in_dim), lambda: (0, 0)),
            pl.BlockSpec((in_dim, out_dim), lambda: (0, 0)),
            pl.BlockSpec((1, out_dim), lambda: (0, 0)),
            pl.BlockSpec((1, out_dim), lambda: (0, 0)),
            pl.BlockSpec(memory_space=pltpu.MemorySpace.HBM),
        ],
        out_specs=pl.BlockSpec(memory_space=pltpu.MemorySpace.HBM),
        out_shape=jax.ShapeDtypeStruct((n, out_dim), jnp.float32),
        scratch_shapes=[
            pltpu.VMEM((batch * n, out_dim), jnp.float32),
            pltpu.VMEM((nbuf, block_m, n), jnp.float32),
            pltpu.VMEM((n, out_dim), jnp.float32),
            pltpu.SemaphoreType.DMA((nbuf,)),
            pltpu.SemaphoreType.DMA((out_chunks,)),
        ],
        compiler_params=pltpu.CompilerParams(
            vmem_limit_bytes=64 * 1024 * 1024),
        interpret=interpret,
    )(hf, W, gamma, beta, A_normalized)
    return out.reshape(batch, n, out_dim)


def kernel(H, A_normalized, W, bn_gamma, bn_beta):
    return _gcn(H, A_normalized, W, bn_gamma, bn_beta)


# speedup vs baseline: 1.0119x; 1.0119x over previous
import functools

import jax
import jax.numpy as jnp
from jax.experimental import pallas as pl
from jax.experimental.pallas import tpu as pltpu

_BN_EPS = 1e-5


def _make_kernel(n, block_m, nbuf):
    nblk = n // block_m

    def _kernel(hf_ref, w_ref, gamma_ref, beta_ref, a_hbm, out_ref,
                ht_scratch, abuf, sems):
        def copy_in(i):
            return pltpu.make_async_copy(
                a_hbm.at[pl.ds(i * block_m, block_m), :],
                abuf.at[i % nbuf],
                sems.at[i % nbuf])

        for i in range(min(nbuf, nblk)):
            copy_in(i).start()

        ht = jnp.dot(hf_ref[...], w_ref[...],
                     preferred_element_type=jnp.float32)
        mean = jnp.mean(ht, axis=0, keepdims=True)
        var = jnp.mean(jnp.square(ht - mean), axis=0, keepdims=True)
        inv = jax.lax.rsqrt(var + _BN_EPS)
        htn = (ht - mean) * inv * gamma_ref[...] + beta_ref[...]
        ht_scratch[...] = jnp.maximum(htn, 0.0)

        for i in range(nblk):
            copy_in(i).wait()
            out_ref[pl.ds(i * block_m, block_m), :] = jnp.dot(
                abuf[i % nbuf], ht_scratch[...],
                preferred_element_type=jnp.float32)
            if i + nbuf < nblk:
                copy_in(i + nbuf).start()

    return _kernel


@functools.partial(jax.jit, static_argnames=("block_m", "nbuf", "interpret"))
def _gcn(H, A_normalized, W, bn_gamma, bn_beta, block_m=80, nbuf=12,
         interpret=False):
    batch, n, in_dim = H.shape
    out_dim = W.shape[1]
    hf = H.reshape(batch * n, in_dim)
    gamma = bn_gamma.reshape(1, out_dim)
    beta = bn_beta.reshape(1, out_dim)

    out = pl.pallas_call(
        _make_kernel(n, block_m, nbuf),
        in_specs=[
            pl.BlockSpec((batch * n, in_dim), lambda: (0, 0)),
            pl.BlockSpec((in_dim, out_dim), lambda: (0, 0)),
            pl.BlockSpec((1, out_dim), lambda: (0, 0)),
            pl.BlockSpec((1, out_dim), lambda: (0, 0)),
            pl.BlockSpec(memory_space=pltpu.MemorySpace.HBM),
        ],
        out_specs=pl.BlockSpec((n, out_dim), lambda: (0, 0)),
        out_shape=jax.ShapeDtypeStruct((n, out_dim), jnp.float32),
        scratch_shapes=[
            pltpu.VMEM((batch * n, out_dim), jnp.float32),
            pltpu.VMEM((nbuf, block_m, n), jnp.float32),
            pltpu.SemaphoreType.DMA((nbuf,)),
        ],
        compiler_params=pltpu.CompilerParams(
            vmem_limit_bytes=64 * 1024 * 1024),
        interpret=interpret,
    )(hf, W, gamma, beta, A_normalized)
    return out.reshape(batch, n, out_dim)


def kernel(H, A_normalized, W, bn_gamma, bn_beta):
    return _gcn(H, A_normalized, W, bn_gamma, bn_beta)
